# baseline (speedup 1.0000x reference)
import jax
import jax.numpy as jnp
from jax import lax
from jax.experimental import pallas as pl
from jax.experimental.pallas import tpu as pltpu

_B, _M, _N, _D = 16, 8192, 8192, 256
_CH = 4096
_NCH = (_B * _M) // _CH
_NB = 4
_CPB = _M // _CH


def _body(idx_ref, num_ref, ishape_ref, pts_ref, wt_ref, emb_ref, out_ref,
          pwin, g_v, b0, b1, b2, b3, psem,
          si0, si1, si2, si3, so0, so1, so2, so3):
    bufs = (b0, b1, b2, b3)
    isems = (si0, si1, si2, si3)
    osems = (so0, so1, so2, so3)

    rs = []
    for b in range(_B):
        off5 = (b * _N + idx_ref[b]) * 5
        base = jnp.minimum((off5 // 128) * 128, _B * _N * 5 - 256)
        rs.append(off5 - base)
        pltpu.make_async_copy(pts_ref.at[pl.ds(base, 256)], pwin.at[b],
                              psem).start()
    for b in range(_B):
        pltpu.make_async_copy(pts_ref.at[pl.ds(0, 256)], pwin.at[b],
                              psem).wait()

    hf = ishape_ref[2].astype(jnp.float32)
    wf = ishape_ref[3].astype(jnp.float32)
    kp_scale = jnp.maximum(wf, hf) * 0.7
    len_scale = jnp.sqrt(wf * wf + hf * hf) * 0.7

    lane = lax.broadcasted_iota(jnp.int32, (1, 256), 1)
    for b in range(_B):
        r = rs[b]
        rowv = pwin[pl.ds(b, 1), :]
        x = jnp.sum(jnp.where(lane == r, rowv, 0.0))
        y = jnp.sum(jnp.where(lane == r + 1, rowv, 0.0))
        ln = jnp.sum(jnp.where(lane == r + 3, rowv, 0.0))
        an = jnp.sum(jnp.where(lane == r + 4, rowv, 0.0))
        nx = (x - wf * 0.5) / kp_scale
        ny = (y - hf * 0.5) / kp_scale
        na = (an - 45.0) / (180.0 * 0.7)
        nl = (ln - len_scale * 0.5) / len_scale
        g_v[pl.ds(b, 1), :] = (nx * wt_ref[0:1, :] + ny * wt_ref[1:2, :]
                               + na * wt_ref[2:3, :] + nl * wt_ref[3:4, :])

    def start_in(ci, k):
        pltpu.make_async_copy(emb_ref.at[pl.ds(ci * _CH, _CH), :], bufs[k],
                              isems[k]).start()

    def start_out(ci, k):
        pltpu.make_async_copy(bufs[k], out_ref.at[pl.ds(ci * _CH, _CH), :],
                              osems[k]).start()

    def wait_in(k):
        pltpu.make_async_copy(emb_ref.at[pl.ds(0, _CH), :], bufs[k],
                              isems[k]).wait()

    def wait_out(k):
        pltpu.make_async_copy(bufs[k], out_ref.at[pl.ds(0, _CH), :],
                              osems[k]).wait()

    start_in(0, 0)
    start_in(1, 1)

    def group(t, carry):
        for s in range(_NB):
            ci = t * _NB + s
            b = ci // _CPB
            kadd = num_ref[b] - (ci % _CPB) * _CH
            grow = g_v[pl.ds(b, 1), :]
            wait_in(s)
            buf = bufs[s]
            for j in range(4):
                rows = _CH // 4
                row = (lax.broadcasted_iota(jnp.int32, (rows, 1), 0)
                       + j * rows)
                sl = pl.ds(j * rows, rows)
                buf[sl, :] = buf[sl, :] + jnp.where(row < kadd, grow, 0.0)
            start_out(ci, s)
            nxt = ci + 2
            kn = (s + 2) % _NB

            @pl.when(nxt < _NCH)
            def _():
                @pl.when(nxt >= _NB)
                def _():
                    wait_out(kn)
                start_in(nxt, kn)
        return carry

    lax.fori_loop(0, _NCH // _NB, group, 0)
    for k in range(_NB):
        wait_out(k)


@jax.jit
def kernel(emb, num, pts, indeces, image_shape, W):
    num = num.astype(jnp.int32)
    indeces = indeces.astype(jnp.int32)
    image_shape = image_shape.astype(jnp.int32)
    wt = W.T

    emb2 = emb.reshape(_B * _M, _D)
    pts1 = pts.reshape(_B * _N * 5)

    out2 = pl.pallas_call(
        _body,
        grid_spec=pltpu.PrefetchScalarGridSpec(
            num_scalar_prefetch=3,
            grid=(1,),
            in_specs=[
                pl.BlockSpec(memory_space=pl.ANY),
                pl.BlockSpec((4, _D), lambda i, idx, n, s: (0, 0)),
                pl.BlockSpec(memory_space=pl.ANY),
            ],
            out_specs=pl.BlockSpec(memory_space=pl.ANY),
            scratch_shapes=[
                pltpu.VMEM((_B, 256), jnp.float32),
                pltpu.VMEM((_B, _D), jnp.float32),
            ]
            + [pltpu.VMEM((_CH, _D), jnp.float32)] * _NB
            + [pltpu.SemaphoreType.DMA] * (1 + 2 * _NB),
        ),
        out_shape=jax.ShapeDtypeStruct((_B * _M, _D), emb.dtype),
    )(indeces, num, image_shape, pts1, wt, emb2)
    return out2.reshape(_B, _M, _D)


# speedup vs baseline: 1.4528x; 1.4528x over previous
import jax
import jax.numpy as jnp
from jax.experimental import pallas as pl
from jax.experimental.pallas import tpu as pltpu

_B, _M, _N, _D = 16, 8192, 8192, 256
_BM = 8192
_PR = 8


def _body(idx_ref, num_ref, ishape_ref, pts_ref, wt_ref, emb_ref, out_ref):
    b = pl.program_id(0)
    j = pl.program_id(1)

    hf = ishape_ref[2].astype(jnp.float32)
    wf = ishape_ref[3].astype(jnp.float32)
    kp_scale = jnp.maximum(wf, hf) * 0.7
    max_len = jnp.sqrt(wf * wf + hf * hf)
    len_scale = max_len * 0.7

    r = idx_ref[b] % _PR
    x = pts_ref[0, r, 0]
    y = pts_ref[0, r, 1]
    ln = pts_ref[0, r, 3]
    an = pts_ref[0, r, 4]

    nx = (x - wf * 0.5) / kp_scale
    ny = (y - hf * 0.5) / kp_scale
    na = (an - 45.0) / (180.0 * 0.7)
    nl = (ln - len_scale * 0.5) / len_scale

    g = (nx * wt_ref[0:1, :] + ny * wt_ref[1:2, :]
         + na * wt_ref[2:3, :] + nl * wt_ref[3:4, :])

    row = jax.lax.broadcasted_iota(jnp.int32, (_BM, 1), 0) + j * _BM
    mask = row < num_ref[b]
    out_ref[0] = emb_ref[0] + jnp.where(mask, g, 0.0)


@jax.jit
def kernel(emb, num, pts, indeces, image_shape, W):
    num = num.astype(jnp.int32)
    indeces = indeces.astype(jnp.int32)
    image_shape = image_shape.astype(jnp.int32)
    wt = W.T

    grid = (_B, _M // _BM)
    return pl.pallas_call(
        _body,
        grid_spec=pltpu.PrefetchScalarGridSpec(
            num_scalar_prefetch=3,
            grid=grid,
            in_specs=[
                pl.BlockSpec((1, _PR, 5),
                             lambda b, j, idx, n, s: (b, idx[b] // _PR, 0)),
                pl.BlockSpec((4, _D), lambda b, j, idx, n, s: (0, 0)),
                pl.BlockSpec((1, _BM, _D), lambda b, j, idx, n, s: (b, j, 0)),
            ],
            out_specs=pl.BlockSpec((1, _BM, _D),
                                   lambda b, j, idx, n, s: (b, j, 0)),
        ),
        out_shape=jax.ShapeDtypeStruct((_B, _M, _D), emb.dtype),
        compiler_params=pltpu.CompilerParams(
            dimension_semantics=("parallel", "parallel"),
        ),
    )(indeces, num, image_shape, pts, wt, emb)
